# Initial kernel scaffold; baseline (speedup 1.0000x reference)
import jax
import jax.numpy as jnp
from jax.experimental import pallas as pl


def kernel(x, edge_index, edge_attr, W1l, b1l, W1r, b1r, W2l, b2l, W2r, b2r, Wg, bg):
    raise NotImplementedError("write your pallas kernel here")


# speedup vs baseline: 10.2681x; 10.2681x over previous
import functools
import jax
import jax.numpy as jnp
from jax import lax
from jax.experimental import pallas as pl
from jax.experimental.pallas import tpu as pltpu
from jax.experimental.pallas import tpu_sc as plsc

NN = 50000
NEG = 0.01
N2 = 50048
RPT = N2 // 16
EE = 800000
E2 = 802816
CH = 128
JUNK = NN
BLK = 3128
F32 = jnp.float32
I32 = jnp.int32

_MESH = plsc.VectorSubcoreMesh(core_axis_name="c", subcore_axis_name="s")
_SC_PARAMS = pltpu.CompilerParams(use_tc_tiling_on_sc=False)


def _deg_body(src_h, dst_h, attr_h, z1_h, outw_h, outa_h, acc_w, acc_a, sbuf,
              dbuf, abuf, wbuf, hop):
    c = lax.axis_index("c")
    s = lax.axis_index("s")
    lo = s * RPT
    pltpu.sync_copy(z1_h, hop)
    pltpu.sync_copy(hop, acc_w.at[pl.ds(lo, RPT)])
    pltpu.sync_copy(hop, acc_a.at[pl.ds(lo, RPT)])
    plsc.subcore_barrier()
    epw = E2 // 32
    e0 = (c * 16 + s) * epw

    def body(i, carry):
        base = e0 + i * CH
        pltpu.sync_copy(src_h.at[pl.ds(base, CH)], sbuf)
        pltpu.sync_copy(dst_h.at[pl.ds(base, CH)], dbuf)
        pltpu.sync_copy(attr_h.at[pl.ds(base, CH)], abuf)
        one = jnp.ones((16,), I32)
        zero = jnp.zeros((16,), I32)
        for g in range(CH // 16):
            s16 = sbuf[pl.ds(g * 16, 16)]
            d16 = dbuf[pl.ds(g * 16, 16)]
            a16 = abuf[pl.ds(g * 16, 16)]
            ls = jnp.where((s16 % 100) < 50, one, zero)
            ld = jnp.where((d16 % 100) < 50, one, zero)
            df = ls - ld
            w16 = jnp.where(df * df < 1, a16, jnp.zeros((16,), F32))
            wbuf[pl.ds(g * 16, 16)] = w16
        pltpu.sync_copy(wbuf, acc_w.at[dbuf], add=True)
        pltpu.sync_copy(abuf, acc_a.at[dbuf], add=True)
        return carry

    lax.fori_loop(0, epw // CH, body, 0)
    plsc.subcore_barrier()
    pltpu.sync_copy(acc_w.at[pl.ds(lo, RPT)], hop)
    pltpu.sync_copy(hop, outw_h.at[pl.ds(c * N2 + lo, RPT)])
    pltpu.sync_copy(acc_a.at[pl.ds(lo, RPT)], hop)
    pltpu.sync_copy(hop, outa_h.at[pl.ds(c * N2 + lo, RPT)])


def _degrees(src, dst, attr, z1):
    return pl.kernel(
        _deg_body,
        out_type=(jax.ShapeDtypeStruct((2 * N2,), F32),
                  jax.ShapeDtypeStruct((2 * N2,), F32)),
        mesh=_MESH,
        compiler_params=_SC_PARAMS,
        scratch_types=[
            pltpu.VMEM_SHARED((N2,), F32),
            pltpu.VMEM_SHARED((N2,), F32),
            pltpu.VMEM((CH,), I32),
            pltpu.VMEM((CH,), I32),
            pltpu.VMEM((CH,), F32),
            pltpu.VMEM((CH,), F32),
            pltpu.VMEM((RPT,), F32),
        ],
    )(src, dst, attr, z1)


def _edge_body(masked, nsplit, src_h, dst_h, attr_h, ztab_h, ytab_h,
               out_h, acc, sbuf, dbuf, abuf, sibuf, rows, hop):
    c = lax.axis_index("c")
    s = lax.axis_index("s")
    lo = s * RPT
    epw = E2 // 16
    e0 = s * epw

    for p in range(nsplit // 2):
        slot = 2 * p + c
        pltpu.sync_copy(ztab_h, hop)
        pltpu.sync_copy(hop, acc.at[pl.ds(lo, RPT)])
        plsc.subcore_barrier()

        def body(i, carry):
            base = e0 + i * CH
            pltpu.sync_copy(src_h.at[pl.ds(base, CH)], sbuf)
            pltpu.sync_copy(dst_h.at[pl.ds(base, CH)], dbuf)
            pltpu.sync_copy(attr_h.at[pl.ds(base, CH)], abuf)
            ws = []
            one = jnp.ones((16,), I32)
            zero = jnp.zeros((16,), I32)
            for g in range(CH // 16):
                s16 = sbuf[pl.ds(g * 16, 16)]
                a16 = abuf[pl.ds(g * 16, 16)]
                if masked:
                    d16 = dbuf[pl.ds(g * 16, 16)]
                    ls = jnp.where((s16 % 100) < 50, one, zero)
                    ld = jnp.where((d16 % 100) < 50, one, zero)
                    df = ls - ld
                    w16 = jnp.where(df * df < 1, a16, jnp.zeros((16,), F32))
                else:
                    w16 = a16
                ws.append(w16)
                sibuf[pl.ds(g * 16, 16)] = s16 * nsplit + slot
            pltpu.sync_copy(ytab_h.at[sibuf], rows)
            for g in range(CH // 16):
                w16 = ws[g]
                for j in range(16):
                    e = g * 16 + j
                    wj = w16[j]
                    rows[e, pl.ds(0, 16)] = rows[e, pl.ds(0, 16)] * wj
            pltpu.sync_copy(rows, acc.at[dbuf], add=True)
            return carry

        lax.fori_loop(0, epw // CH, body, 0)
        plsc.subcore_barrier()
        pltpu.sync_copy(acc.at[pl.ds(lo, RPT)], hop)
        pltpu.sync_copy(hop, out_h.at[pl.ds(slot * N2 + lo, RPT)])
        if p + 1 < nsplit // 2:
            plsc.subcore_barrier()


def _edge_pass(masked, nsplit, src, dst, attr, ztab, ytab):
    return pl.kernel(
        functools.partial(_edge_body, masked, nsplit),
        out_type=jax.ShapeDtypeStruct((nsplit * N2, 16), F32),
        mesh=_MESH,
        compiler_params=_SC_PARAMS,
        scratch_types=[
            pltpu.VMEM_SHARED((N2, 16), F32),
            pltpu.VMEM((CH,), I32),
            pltpu.VMEM((CH,), I32),
            pltpu.VMEM((CH,), F32),
            pltpu.VMEM((CH,), I32),
            pltpu.VMEM((CH, 16), F32),
            pltpu.VMEM((RPT, 16), F32),
        ],
    )(src, dst, attr, ztab, ytab)


def _leaky(v):
    return jnp.maximum(v, NEG * v)


def _row_left(i):
    rows = i * BLK + lax.broadcasted_iota(I32, (BLK, 1), 0)
    return (rows % 100) < 50, rows


def _p0_body(x_ref, wl_ref, wr_ref, dg_ref, y0_ref, y1_ref, y2_ref, y3_ref,
             dsel_ref, dg_out):
    i = pl.program_id(0)
    left, _ = _row_left(i)
    dsum = dg_ref[0] + dg_ref[1]
    deg_sel = dsum[:, 0:1] + 1.0
    deg_g = dsum[:, 1:2] + 1.0
    dinv_sel = jnp.where(deg_sel > 0, lax.rsqrt(jnp.abs(deg_sel) + 1e-30), 0.0)
    dinv_g = jnp.where(deg_g > 0, lax.rsqrt(jnp.abs(deg_g) + 1e-30), 0.0)
    xb = x_ref[...]
    xw = jnp.where(left, jnp.dot(xb, wl_ref[...], preferred_element_type=F32),
                   jnp.dot(xb, wr_ref[...], preferred_element_type=F32))
    y = dinv_sel * xw
    y0_ref[...] = y[:, 0:16]
    y1_ref[...] = y[:, 16:32]
    y2_ref[...] = y[:, 32:48]
    y3_ref[...] = y[:, 48:64]
    dsel_ref[...] = dinv_sel
    dg_out[...] = dinv_g


def _p0(x2, w1l2, w1r2, degp):
    return pl.pallas_call(
        _p0_body,
        grid=(16,),
        in_specs=[
            pl.BlockSpec((BLK, 128), lambda i: (i, 0)),
            pl.BlockSpec((128, 64), lambda i: (0, 0)),
            pl.BlockSpec((128, 64), lambda i: (0, 0)),
            pl.BlockSpec((2, BLK, 2), lambda i: (0, i, 0)),
        ],
        out_specs=[pl.BlockSpec((BLK, 16), lambda i: (i, 0))] * 4
        + [pl.BlockSpec((BLK, 1), lambda i: (i, 0))] * 2,
        out_shape=[jax.ShapeDtypeStruct((N2, 16), F32)] * 4
        + [jax.ShapeDtypeStruct((N2, 1), F32)] * 2,
    )(x2, w1l2, w1r2, degp)


def _p1_body(a_ref, ya_ref, yb_ref, yc_ref, yd_ref, d_ref, bl_ref, br_ref,
             w2l_ref, w2r_ref, y2a_ref, y2b_ref):
    i = pl.program_id(0)
    left, _ = _row_left(i)
    acc = jnp.concatenate([a_ref[0], a_ref[1], a_ref[2], a_ref[3]], axis=1)
    y = jnp.concatenate([ya_ref[...], yb_ref[...], yc_ref[...], yd_ref[...]],
                        axis=1)
    dinv = d_ref[...]
    bsel = jnp.where(left, bl_ref[...], br_ref[...])
    h1 = _leaky(dinv * (acc + y) + bsel)
    xw = jnp.where(left,
                   jnp.dot(h1, w2l_ref[...], preferred_element_type=F32),
                   jnp.dot(h1, w2r_ref[...], preferred_element_type=F32))
    y2 = dinv * xw
    y2a_ref[...] = y2[:, 0:16]
    y2b_ref[...] = y2[:, 16:32]


def _p1(acc1, ys, dsel, b1l, b1r, w2l2, w2r2):
    return pl.pallas_call(
        _p1_body,
        grid=(16,),
        in_specs=[
            pl.BlockSpec((4, BLK, 16), lambda i: (0, i, 0)),
            pl.BlockSpec((BLK, 16), lambda i: (i, 0)),
            pl.BlockSpec((BLK, 16), lambda i: (i, 0)),
            pl.BlockSpec((BLK, 16), lambda i: (i, 0)),
            pl.BlockSpec((BLK, 16), lambda i: (i, 0)),
            pl.BlockSpec((BLK, 1), lambda i: (i, 0)),
            pl.BlockSpec((1, 64), lambda i: (0, 0)),
            pl.BlockSpec((1, 64), lambda i: (0, 0)),
            pl.BlockSpec((64, 32), lambda i: (0, 0)),
            pl.BlockSpec((64, 32), lambda i: (0, 0)),
        ],
        out_specs=[pl.BlockSpec((BLK, 16), lambda i: (i, 0))] * 2,
        out_shape=[jax.ShapeDtypeStruct((N2, 16), F32)] * 2,
    )(acc1, ys[0], ys[1], ys[2], ys[3], dsel, b1l, b1r, w2l2, w2r2)


def _p2_body(a_ref, ya_ref, yb_ref, dsel_ref, dg_ref, bl_ref, br_ref, wg_ref,
             y3a_ref, y3b_ref):
    i = pl.program_id(0)
    left, _ = _row_left(i)
    acc = jnp.concatenate([a_ref[0], a_ref[1]], axis=1)
    y = jnp.concatenate([ya_ref[...], yb_ref[...]], axis=1)
    dinv = dsel_ref[...]
    bsel = jnp.where(left, bl_ref[...], br_ref[...])
    h2 = _leaky(dinv * (acc + y) + bsel)
    xw = jnp.dot(h2, wg_ref[...], preferred_element_type=F32)
    y3 = dg_ref[...] * xw
    y3a_ref[...] = y3[:, 0:16]
    y3b_ref[...] = y3[:, 16:32]


def _p2(acc2, y2s, dsel, dg, b2l2, b2r2, wg2):
    return pl.pallas_call(
        _p2_body,
        grid=(16,),
        in_specs=[
            pl.BlockSpec((2, BLK, 16), lambda i: (0, i, 0)),
            pl.BlockSpec((BLK, 16), lambda i: (i, 0)),
            pl.BlockSpec((BLK, 16), lambda i: (i, 0)),
            pl.BlockSpec((BLK, 1), lambda i: (i, 0)),
            pl.BlockSpec((BLK, 1), lambda i: (i, 0)),
            pl.BlockSpec((1, 32), lambda i: (0, 0)),
            pl.BlockSpec((1, 32), lambda i: (0, 0)),
            pl.BlockSpec((32, 32), lambda i: (0, 0)),
        ],
        out_specs=[pl.BlockSpec((BLK, 16), lambda i: (i, 0))] * 2,
        out_shape=[jax.ShapeDtypeStruct((N2, 16), F32)] * 2,
    )(acc2, y2s[0], y2s[1], dsel, dg, b2l2, b2r2, wg2)


def _p3_body(a_ref, ya_ref, yb_ref, dg_ref, bg_ref, out_ref):
    i = pl.program_id(0)
    _, rows = _row_left(i)
    acc = jnp.concatenate([a_ref[0], a_ref[1]], axis=1)
    y = jnp.concatenate([ya_ref[...], yb_ref[...]], axis=1)
    h3 = _leaky(dg_ref[...] * (acc + y) + bg_ref[...])
    h3 = jnp.where(rows < NN, h3, 0.0)
    part = jnp.sum(h3, axis=0, keepdims=True) * (1.0 / NN)

    @pl.when(i == 0)
    def _():
        out_ref[...] = jnp.zeros_like(out_ref)

    out_ref[0:1, :] += part


def _p3(acc3, y3s, dg, bg2):
    return pl.pallas_call(
        _p3_body,
        grid=(16,),
        in_specs=[
            pl.BlockSpec((2, BLK, 16), lambda i: (0, i, 0)),
            pl.BlockSpec((BLK, 16), lambda i: (i, 0)),
            pl.BlockSpec((BLK, 16), lambda i: (i, 0)),
            pl.BlockSpec((BLK, 1), lambda i: (i, 0)),
            pl.BlockSpec((1, 32), lambda i: (0, 0)),
        ],
        out_specs=pl.BlockSpec((8, 32), lambda i: (0, 0)),
        out_shape=jax.ShapeDtypeStruct((8, 32), F32),
    )(acc3, y3s[0], y3s[1], dg, bg2)


def kernel(x, edge_index, edge_attr, W1l, b1l, W1r, b1r, W2l, b2l, W2r, b2r,
           Wg, bg):
    src = jnp.concatenate([edge_index[0].astype(I32),
                           jnp.zeros((E2 - EE,), I32)])
    dst = jnp.concatenate([edge_index[1].astype(I32),
                           jnp.full((E2 - EE,), JUNK, I32)])
    attr = jnp.concatenate([edge_attr, jnp.zeros((E2 - EE,), F32)])

    x2 = jnp.zeros((N2, 128), F32).at[:NN, :100].set(x)
    w1l2 = jnp.zeros((128, 64), F32).at[:100, :].set(W1l)
    w1r2 = jnp.zeros((128, 64), F32).at[:100, :].set(W1r)
    w2l2 = jnp.zeros((64, 32), F32).at[:, :20].set(W2l)
    w2r2 = jnp.zeros((64, 32), F32).at[:, :20].set(W2r)
    wg2 = jnp.zeros((32, 32), F32).at[:20, :20].set(Wg)
    b1l2 = b1l.reshape(1, 64)
    b1r2 = b1r.reshape(1, 64)
    b2l2 = jnp.zeros((1, 32), F32).at[0, :20].set(b2l)
    b2r2 = jnp.zeros((1, 32), F32).at[0, :20].set(b2r)
    bg2 = jnp.zeros((1, 32), F32).at[0, :20].set(bg)

    z1 = jnp.zeros((RPT,), F32)
    ztab = jnp.zeros((RPT, 16), F32)

    degw, dega = _degrees(src, dst, attr, z1)
    degp = jnp.stack([degw.reshape(2, N2), dega.reshape(2, N2)], axis=2)
    y1s = _p0(x2, w1l2, w1r2, degp)
    ys, dsel, dg = y1s[:4], y1s[4], y1s[5]

    y1tab = jnp.stack(ys, axis=1).reshape(4 * N2, 16)
    acc1 = _edge_pass(True, 4, src, dst, attr, ztab, y1tab).reshape(
        4, N2, 16)
    y2s = _p1(acc1, ys, dsel, b1l2, b1r2, w2l2, w2r2)

    y2tab = jnp.stack(y2s, axis=1).reshape(2 * N2, 16)
    acc2 = _edge_pass(True, 2, src, dst, attr, ztab, y2tab).reshape(
        2, N2, 16)
    y3s = _p2(acc2, y2s, dsel, dg, b2l2, b2r2, wg2)

    y3tab = jnp.stack(y3s, axis=1).reshape(2 * N2, 16)
    acc3 = _edge_pass(False, 2, src, dst, attr, ztab, y3tab).reshape(
        2, N2, 16)
    out = _p3(acc3, y3s, dg, bg2)
    return out[0:1, :20]


# speedup vs baseline: 13.6629x; 1.3306x over previous
import functools
import jax
import jax.numpy as jnp
from jax import lax
from jax.experimental import pallas as pl
from jax.experimental.pallas import tpu as pltpu
from jax.experimental.pallas import tpu_sc as plsc

NN = 50000
NEG = 0.01
N2 = 50048
RPT = N2 // 16
EE = 800000
E2 = 802816
CH = 128
JUNK = NN
BLK = 3128
F32 = jnp.float32
I32 = jnp.int32

_MESH = plsc.VectorSubcoreMesh(core_axis_name="c", subcore_axis_name="s")
_SC_PARAMS = pltpu.CompilerParams(use_tc_tiling_on_sc=False)


def _deg_body(src_h, dst_h, attr_h, z1_h, outw_h, outa_h, acc_w, acc_a, sbuf,
              dbuf, abuf, wbuf, hop):
    c = lax.axis_index("c")
    s = lax.axis_index("s")
    lo = s * RPT
    pltpu.sync_copy(z1_h, hop)
    pltpu.sync_copy(hop, acc_w.at[pl.ds(lo, RPT)])
    pltpu.sync_copy(hop, acc_a.at[pl.ds(lo, RPT)])
    plsc.subcore_barrier()
    epw = E2 // 32
    e0 = (c * 16 + s) * epw

    def body(i, carry):
        base = e0 + i * CH
        pltpu.sync_copy(src_h.at[pl.ds(base, CH)], sbuf)
        pltpu.sync_copy(dst_h.at[pl.ds(base, CH)], dbuf)
        pltpu.sync_copy(attr_h.at[pl.ds(base, CH)], abuf)
        one = jnp.ones((16,), I32)
        zero = jnp.zeros((16,), I32)
        for g in range(CH // 16):
            s16 = sbuf[pl.ds(g * 16, 16)]
            d16 = dbuf[pl.ds(g * 16, 16)]
            a16 = abuf[pl.ds(g * 16, 16)]
            ls = jnp.where((s16 % 100) < 50, one, zero)
            ld = jnp.where((d16 % 100) < 50, one, zero)
            df = ls - ld
            w16 = jnp.where(df * df < 1, a16, jnp.zeros((16,), F32))
            wbuf[pl.ds(g * 16, 16)] = w16
        pltpu.sync_copy(wbuf, acc_w.at[dbuf], add=True)
        pltpu.sync_copy(abuf, acc_a.at[dbuf], add=True)
        return carry

    lax.fori_loop(0, epw // CH, body, 0)
    plsc.subcore_barrier()
    pltpu.sync_copy(acc_w.at[pl.ds(lo, RPT)], hop)
    pltpu.sync_copy(hop, outw_h.at[pl.ds(c * N2 + lo, RPT)])
    pltpu.sync_copy(acc_a.at[pl.ds(lo, RPT)], hop)
    pltpu.sync_copy(hop, outa_h.at[pl.ds(c * N2 + lo, RPT)])


def _degrees(src, dst, attr, z1):
    return pl.kernel(
        _deg_body,
        out_type=(jax.ShapeDtypeStruct((2 * N2,), F32),
                  jax.ShapeDtypeStruct((2 * N2,), F32)),
        mesh=_MESH,
        compiler_params=_SC_PARAMS,
        scratch_types=[
            pltpu.VMEM_SHARED((N2,), F32),
            pltpu.VMEM_SHARED((N2,), F32),
            pltpu.VMEM((CH,), I32),
            pltpu.VMEM((CH,), I32),
            pltpu.VMEM((CH,), F32),
            pltpu.VMEM((CH,), F32),
            pltpu.VMEM((RPT,), F32),
        ],
    )(src, dst, attr, z1)


NCH = E2 // 16 // CH


def _edge_body(masked, nsplit, ei_h, ea_h, ztab_h, ytab_h, out_h, acc,
               eb0, eb1, ab0, ab1, si0, si1, w0, w1, dx0, dx1, r0, r1, hop,
               ld0, ld1, gs0, gs1, ss0, ss1):
    c = lax.axis_index("c")
    s = lax.axis_index("s")
    lo = s * RPT
    c0 = s * NCH
    EB = (eb0, eb1)
    AB = (ab0, ab1)
    SI = (si0, si1)
    WV = (w0, w1)
    DX = (dx0, dx1)
    RR = (r0, r1)
    LD = (ld0, ld1)
    GS = (gs0, gs1)
    SS = (ss0, ss1)

    def fire_load(k, b):
        pltpu.async_copy(ei_h.at[c0 + k], EB[b], LD[b])
        pltpu.async_copy(ea_h.at[c0 + k], AB[b], LD[b])

    def wait_load(b):
        pltpu.make_async_copy(ei_h.at[c0], EB[b], LD[b]).wait()
        pltpu.make_async_copy(ea_h.at[c0], AB[b], LD[b]).wait()

    def compute(b, slot):
        eb = EB[b]
        one = jnp.ones((16,), I32)
        zero = jnp.zeros((16,), I32)
        for g in range(CH // 16):
            s16 = eb[0, pl.ds(g * 16, 16)]
            d16 = eb[1, pl.ds(g * 16, 16)]
            a16 = AB[b][pl.ds(g * 16, 16)]
            if masked:
                lsv = jnp.where((s16 % 100) < 50, one, zero)
                ldv = jnp.where((d16 % 100) < 50, one, zero)
                df = lsv - ldv
                w16 = jnp.where(df * df < 1, a16, jnp.zeros((16,), F32))
            else:
                w16 = a16
            WV[b][pl.ds(g * 16, 16)] = w16
            SI[b][pl.ds(g * 16, 16)] = s16 * nsplit + slot
            DX[b][pl.ds(g * 16, 16)] = d16

    def fire_gather(b):
        pltpu.async_copy(ytab_h.at[SI[b]], RR[b], GS[b])

    def wait_gather(b):
        pltpu.make_async_copy(ytab_h.at[SI[b]], RR[b], GS[b]).wait()

    def scale(b):
        r = RR[b]
        for g in range(CH // 16):
            w16 = WV[b][pl.ds(g * 16, 16)]
            for j in range(16):
                e = g * 16 + j
                wj = lax.gather(
                    w16, jnp.full((16, 1), j, I32),
                    lax.GatherDimensionNumbers(
                        offset_dims=(), collapsed_slice_dims=(0,),
                        start_index_map=(0,)),
                    (1,), mode=lax.GatherScatterMode.PROMISE_IN_BOUNDS)
                r[e, pl.ds(0, 16)] = r[e, pl.ds(0, 16)] * wj

    def fire_scatter(b):
        pltpu.async_copy(RR[b], acc.at[DX[b]], SS[b], add=True)

    def wait_scatter(b):
        pltpu.make_async_copy(RR[b], acc.at[DX[b]], SS[b]).wait()

    def phase(p, slot):
        pltpu.sync_copy(ztab_h, hop)
        pltpu.sync_copy(hop, acc.at[pl.ds(lo, RPT)])
        plsc.subcore_barrier()
        fire_load(0, 0)
        fire_load(1, 1)
        wait_load(0)
        compute(0, slot)
        fire_gather(0)
        wait_gather(0)
        scale(0)
        fire_scatter(0)
        wait_load(1)
        compute(1, slot)
        fire_gather(1)
        fire_load(2, 0)

        def half(q, b):
            wait_gather(b)
            scale(b)
            fire_scatter(b)
            wait_load(1 - b)
            wait_scatter(1 - b)
            compute(1 - b, slot)
            fire_gather(1 - b)
            fire_load(lax.rem(q + 2, NCH), b)

        def body(t, carry):
            q = 1 + 2 * t
            half(q, 1)
            half(q + 1, 0)
            return carry

        lax.fori_loop(0, (NCH - 2) // 2, body, 0)
        wait_gather(1)
        scale(1)
        fire_scatter(1)
        wait_load(0)
        wait_scatter(0)
        wait_scatter(1)
        plsc.subcore_barrier()
        pltpu.sync_copy(acc.at[pl.ds(lo, RPT)], hop)
        pltpu.sync_copy(hop, out_h.at[pl.ds(slot * N2 + lo, RPT)])

    if nsplit == 2:
        phase(0, c)
    else:
        def phase_body(p, carry):
            phase(p, 2 * p + c)
            plsc.subcore_barrier()
            return carry

        lax.fori_loop(0, nsplit // 2, phase_body, 0)


def _edge_pass(masked, nsplit, ei, ea, ztab, ytab):
    return pl.kernel(
        functools.partial(_edge_body, masked, nsplit),
        out_type=jax.ShapeDtypeStruct((nsplit * N2, 16), F32),
        mesh=_MESH,
        compiler_params=_SC_PARAMS,
        scratch_types=[
            pltpu.VMEM_SHARED((N2, 16), F32),
            pltpu.VMEM((2, CH), I32),
            pltpu.VMEM((2, CH), I32),
            pltpu.VMEM((CH,), F32),
            pltpu.VMEM((CH,), F32),
            pltpu.VMEM((CH,), I32),
            pltpu.VMEM((CH,), I32),
            pltpu.VMEM((CH,), F32),
            pltpu.VMEM((CH,), F32),
            pltpu.VMEM((CH,), I32),
            pltpu.VMEM((CH,), I32),
            pltpu.VMEM((CH, 16), F32),
            pltpu.VMEM((CH, 16), F32),
            pltpu.VMEM((RPT, 16), F32),
            pltpu.SemaphoreType.DMA,
            pltpu.SemaphoreType.DMA,
            pltpu.SemaphoreType.DMA,
            pltpu.SemaphoreType.DMA,
            pltpu.SemaphoreType.DMA,
            pltpu.SemaphoreType.DMA,
        ],
    )(ei, ea, ztab, ytab)


def _leaky(v):
    return jnp.maximum(v, NEG * v)


def _row_left(i):
    rows = i * BLK + lax.broadcasted_iota(I32, (BLK, 1), 0)
    return (rows % 100) < 50, rows


def _p0_body(x_ref, wl_ref, wr_ref, dg_ref, y0_ref, y1_ref, y2_ref, y3_ref,
             dsel_ref, dg_out):
    i = pl.program_id(0)
    left, _ = _row_left(i)
    dsum = dg_ref[0] + dg_ref[1]
    deg_sel = dsum[:, 0:1] + 1.0
    deg_g = dsum[:, 1:2] + 1.0
    dinv_sel = jnp.where(deg_sel > 0, lax.rsqrt(jnp.abs(deg_sel) + 1e-30), 0.0)
    dinv_g = jnp.where(deg_g > 0, lax.rsqrt(jnp.abs(deg_g) + 1e-30), 0.0)
    xb = x_ref[...]
    xw = jnp.where(left, jnp.dot(xb, wl_ref[...], preferred_element_type=F32),
                   jnp.dot(xb, wr_ref[...], preferred_element_type=F32))
    y = dinv_sel * xw
    y0_ref[...] = y[:, 0:16]
    y1_ref[...] = y[:, 16:32]
    y2_ref[...] = y[:, 32:48]
    y3_ref[...] = y[:, 48:64]
    dsel_ref[...] = dinv_sel
    dg_out[...] = dinv_g


def _p0(x2, w1l2, w1r2, degp):
    return pl.pallas_call(
        _p0_body,
        grid=(16,),
        in_specs=[
            pl.BlockSpec((BLK, 128), lambda i: (i, 0)),
            pl.BlockSpec((128, 64), lambda i: (0, 0)),
            pl.BlockSpec((128, 64), lambda i: (0, 0)),
            pl.BlockSpec((2, BLK, 2), lambda i: (0, i, 0)),
        ],
        out_specs=[pl.BlockSpec((BLK, 16), lambda i: (i, 0))] * 4
        + [pl.BlockSpec((BLK, 1), lambda i: (i, 0))] * 2,
        out_shape=[jax.ShapeDtypeStruct((N2, 16), F32)] * 4
        + [jax.ShapeDtypeStruct((N2, 1), F32)] * 2,
    )(x2, w1l2, w1r2, degp)


def _p1_body(a_ref, ya_ref, yb_ref, yc_ref, yd_ref, d_ref, bl_ref, br_ref,
             w2l_ref, w2r_ref, y2a_ref, y2b_ref):
    i = pl.program_id(0)
    left, _ = _row_left(i)
    acc = jnp.concatenate([a_ref[0], a_ref[1], a_ref[2], a_ref[3]], axis=1)
    y = jnp.concatenate([ya_ref[...], yb_ref[...], yc_ref[...], yd_ref[...]],
                        axis=1)
    dinv = d_ref[...]
    bsel = jnp.where(left, bl_ref[...], br_ref[...])
    h1 = _leaky(dinv * (acc + y) + bsel)
    xw = jnp.where(left,
                   jnp.dot(h1, w2l_ref[...], preferred_element_type=F32),
                   jnp.dot(h1, w2r_ref[...], preferred_element_type=F32))
    y2 = dinv * xw
    y2a_ref[...] = y2[:, 0:16]
    y2b_ref[...] = y2[:, 16:32]


def _p1(acc1, ys, dsel, b1l, b1r, w2l2, w2r2):
    return pl.pallas_call(
        _p1_body,
        grid=(16,),
        in_specs=[
            pl.BlockSpec((4, BLK, 16), lambda i: (0, i, 0)),
            pl.BlockSpec((BLK, 16), lambda i: (i, 0)),
            pl.BlockSpec((BLK, 16), lambda i: (i, 0)),
            pl.BlockSpec((BLK, 16), lambda i: (i, 0)),
            pl.BlockSpec((BLK, 16), lambda i: (i, 0)),
            pl.BlockSpec((BLK, 1), lambda i: (i, 0)),
            pl.BlockSpec((1, 64), lambda i: (0, 0)),
            pl.BlockSpec((1, 64), lambda i: (0, 0)),
            pl.BlockSpec((64, 32), lambda i: (0, 0)),
            pl.BlockSpec((64, 32), lambda i: (0, 0)),
        ],
        out_specs=[pl.BlockSpec((BLK, 16), lambda i: (i, 0))] * 2,
        out_shape=[jax.ShapeDtypeStruct((N2, 16), F32)] * 2,
    )(acc1, ys[0], ys[1], ys[2], ys[3], dsel, b1l, b1r, w2l2, w2r2)


def _p2_body(a_ref, ya_ref, yb_ref, dsel_ref, dg_ref, bl_ref, br_ref, wg_ref,
             y3a_ref, y3b_ref):
    i = pl.program_id(0)
    left, _ = _row_left(i)
    acc = jnp.concatenate([a_ref[0], a_ref[1]], axis=1)
    y = jnp.concatenate([ya_ref[...], yb_ref[...]], axis=1)
    dinv = dsel_ref[...]
    bsel = jnp.where(left, bl_ref[...], br_ref[...])
    h2 = _leaky(dinv * (acc + y) + bsel)
    xw = jnp.dot(h2, wg_ref[...], preferred_element_type=F32)
    y3 = dg_ref[...] * xw
    y3a_ref[...] = y3[:, 0:16]
    y3b_ref[...] = y3[:, 16:32]


def _p2(acc2, y2s, dsel, dg, b2l2, b2r2, wg2):
    return pl.pallas_call(
        _p2_body,
        grid=(16,),
        in_specs=[
            pl.BlockSpec((2, BLK, 16), lambda i: (0, i, 0)),
            pl.BlockSpec((BLK, 16), lambda i: (i, 0)),
            pl.BlockSpec((BLK, 16), lambda i: (i, 0)),
            pl.BlockSpec((BLK, 1), lambda i: (i, 0)),
            pl.BlockSpec((BLK, 1), lambda i: (i, 0)),
            pl.BlockSpec((1, 32), lambda i: (0, 0)),
            pl.BlockSpec((1, 32), lambda i: (0, 0)),
            pl.BlockSpec((32, 32), lambda i: (0, 0)),
        ],
        out_specs=[pl.BlockSpec((BLK, 16), lambda i: (i, 0))] * 2,
        out_shape=[jax.ShapeDtypeStruct((N2, 16), F32)] * 2,
    )(acc2, y2s[0], y2s[1], dsel, dg, b2l2, b2r2, wg2)


def _p3_body(a_ref, ya_ref, yb_ref, dg_ref, bg_ref, out_ref):
    i = pl.program_id(0)
    _, rows = _row_left(i)
    acc = jnp.concatenate([a_ref[0], a_ref[1]], axis=1)
    y = jnp.concatenate([ya_ref[...], yb_ref[...]], axis=1)
    h3 = _leaky(dg_ref[...] * (acc + y) + bg_ref[...])
    h3 = jnp.where(rows < NN, h3, 0.0)
    part = jnp.sum(h3, axis=0, keepdims=True) * (1.0 / NN)

    @pl.when(i == 0)
    def _():
        out_ref[...] = jnp.zeros_like(out_ref)

    out_ref[0:1, :] += part


def _p3(acc3, y3s, dg, bg2):
    return pl.pallas_call(
        _p3_body,
        grid=(16,),
        in_specs=[
            pl.BlockSpec((2, BLK, 16), lambda i: (0, i, 0)),
            pl.BlockSpec((BLK, 16), lambda i: (i, 0)),
            pl.BlockSpec((BLK, 16), lambda i: (i, 0)),
            pl.BlockSpec((BLK, 1), lambda i: (i, 0)),
            pl.BlockSpec((1, 32), lambda i: (0, 0)),
        ],
        out_specs=pl.BlockSpec((8, 32), lambda i: (0, 0)),
        out_shape=jax.ShapeDtypeStruct((8, 32), F32),
    )(acc3, y3s[0], y3s[1], dg, bg2)


def kernel(x, edge_index, edge_attr, W1l, b1l, W1r, b1r, W2l, b2l, W2r, b2r,
           Wg, bg):
    src = jnp.concatenate([edge_index[0].astype(I32),
                           jnp.zeros((E2 - EE,), I32)])
    dst = jnp.concatenate([edge_index[1].astype(I32),
                           jnp.full((E2 - EE,), JUNK, I32)])
    attr = jnp.concatenate([edge_attr, jnp.zeros((E2 - EE,), F32)])

    x2 = jnp.zeros((N2, 128), F32).at[:NN, :100].set(x)
    w1l2 = jnp.zeros((128, 64), F32).at[:100, :].set(W1l)
    w1r2 = jnp.zeros((128, 64), F32).at[:100, :].set(W1r)
    w2l2 = jnp.zeros((64, 32), F32).at[:, :20].set(W2l)
    w2r2 = jnp.zeros((64, 32), F32).at[:, :20].set(W2r)
    wg2 = jnp.zeros((32, 32), F32).at[:20, :20].set(Wg)
    b1l2 = b1l.reshape(1, 64)
    b1r2 = b1r.reshape(1, 64)
    b2l2 = jnp.zeros((1, 32), F32).at[0, :20].set(b2l)
    b2r2 = jnp.zeros((1, 32), F32).at[0, :20].set(b2r)
    bg2 = jnp.zeros((1, 32), F32).at[0, :20].set(bg)

    z1 = jnp.zeros((RPT,), F32)
    ztab = jnp.zeros((RPT, 16), F32)
    ei = jnp.stack([src.reshape(-1, CH), dst.reshape(-1, CH)], axis=1)
    ea = attr.reshape(-1, CH)

    degw, dega = _degrees(src, dst, attr, z1)
    degp = jnp.stack([degw.reshape(2, N2), dega.reshape(2, N2)], axis=2)
    y1s = _p0(x2, w1l2, w1r2, degp)
    ys, dsel, dg = y1s[:4], y1s[4], y1s[5]

    y1tab = jnp.stack(ys, axis=1).reshape(4 * N2, 16)
    acc1 = _edge_pass(True, 4, ei, ea, ztab, y1tab).reshape(
        4, N2, 16)
    y2s = _p1(acc1, ys, dsel, b1l2, b1r2, w2l2, w2r2)

    y2tab = jnp.stack(y2s, axis=1).reshape(2 * N2, 16)
    acc2 = _edge_pass(True, 2, ei, ea, ztab, y2tab).reshape(
        2, N2, 16)
    y3s = _p2(acc2, y2s, dsel, dg, b2l2, b2r2, wg2)

    y3tab = jnp.stack(y3s, axis=1).reshape(2 * N2, 16)
    acc3 = _edge_pass(False, 2, ei, ea, ztab, y3tab).reshape(
        2, N2, 16)
    out = _p3(acc3, y3s, dg, bg2)
    return out[0:1, :20]
